# Initial kernel scaffold; baseline (speedup 1.0000x reference)
import jax
import jax.numpy as jnp
from jax.experimental import pallas as pl


def kernel(x, W, b, ctx_mod, context):
    raise NotImplementedError("write your pallas kernel here")


# speedup vs baseline: 1.2383x; 1.2383x over previous
import functools

import jax
import jax.numpy as jnp
from jax import lax
from jax.experimental import pallas as pl
from jax.experimental.pallas import tpu as pltpu
from jax.experimental.pallas import tpu_sc as plsc

N_RECEP = 4
BATCH = 2048
L = 128
C = 4096
T = N_RECEP * BATCH
AVG_N = 50000.0

TB = 256
NBB = BATCH // TB
GRID = NBB * N_RECEP

NC = 2
NS = 16
NW = NC * NS
ROWS_PER_W = C // NW
TOK_PER_W = T // NW
CHUNK = 128
NCHUNK = TOK_PER_W // CHUNK


def _tc_body(x_ref, ctx_ref, w_ref, b_ref, cm_ref, out_ref, argm_ref,
             cn_ref, mseg_ref):
    i = pl.program_id(0)
    n = i % N_RECEP

    @pl.when(i == 0)
    def _init():
        ctx = ctx_ref[...]
        cnorm = jnp.sqrt(jnp.sum(ctx * ctx, axis=1, keepdims=True))
        cn_ref[...] = ctx / (cnorm + 1e-8)
        seg = lax.dot_general(cm_ref[...], ctx, (((1,), (1,)), ((), ())),
                              preferred_element_type=jnp.float32)
        mseg_ref[...] = jnp.max(seg, axis=0, keepdims=True)

    xs = x_ref[...]
    xnorm = jnp.sqrt(jnp.sum(xs * xs, axis=1, keepdims=True))
    xn = xs / (xnorm + 1e-8)
    sim = lax.dot_general(xn, cn_ref[...], (((1,), (1,)), ((), ())),
                          preferred_element_type=jnp.float32)
    m = jnp.max(sim, axis=1, keepdims=True)
    iota = lax.broadcasted_iota(jnp.int32, sim.shape, 1)
    am = jnp.min(jnp.where(sim == m, iota, jnp.int32(2**30)),
                 axis=1, keepdims=True)
    msel = jnp.max(jnp.where(iota == am, mseg_ref[...], -jnp.inf),
                   axis=1, keepdims=True)
    act = jax.nn.sigmoid(msel)

    rec = jax.nn.gelu(lax.dot_general(xs, w_ref[...], (((1,), (0,)), ((), ())),
                                      preferred_element_type=jnp.float32)
                      + b_ref[...])
    contrib = rec * act * (1.0 / N_RECEP)

    @pl.when(n == 0)
    def _set():
        out_ref[...] = contrib

    @pl.when(n != 0)
    def _acc():
        out_ref[...] += contrib

    argm_ref[...] = am


def _sc_body(ctx_hbm, xs_hbm, idx_hbm, out_hbm, idxv, rows, xsv, sem):
    wid = lax.axis_index("s") * NC + lax.axis_index("c")
    rbase = wid * ROWS_PER_W
    pltpu.sync_copy(ctx_hbm.at[pl.ds(rbase, ROWS_PER_W)], rows)
    pltpu.sync_copy(rows, out_hbm.at[pl.ds(rbase, ROWS_PER_W)])
    plsc.subcore_barrier()
    for j in range(NCHUNK):
        tbase = wid * TOK_PER_W + j * CHUNK
        pltpu.sync_copy(idx_hbm.at[pl.ds(tbase, CHUNK)], idxv)
        pltpu.async_copy(ctx_hbm.at[idxv], rows, sem).wait()
        pltpu.sync_copy(xs_hbm.at[pl.ds(tbase, CHUNK)], xsv)

        def body(r, carry):
            for cc in range(L // 16):
                sl = pl.ds(cc * 16, 16)
                old = rows[r, sl]
                rows[r, sl] = (old * (AVG_N - 1.0) + xsv[r, sl]) / AVG_N
            return carry

        lax.fori_loop(0, CHUNK, body, 0)
        pltpu.async_copy(rows, out_hbm.at[idxv], sem).wait()


def _sc_update(context, xf, idx):
    sc = functools.partial(
        pl.kernel,
        out_type=jax.ShapeDtypeStruct((C, L), jnp.float32),
        mesh=plsc.VectorSubcoreMesh(core_axis_name="c", subcore_axis_name="s",
                                    num_cores=NC, num_subcores=NS),
        scratch_types=[
            pltpu.VMEM((CHUNK,), jnp.int32),
            pltpu.VMEM((CHUNK, L), jnp.float32),
            pltpu.VMEM((CHUNK, L), jnp.float32),
            pltpu.SemaphoreType.DMA,
        ],
    )(_sc_body)
    return sc(context, xf, idx)


def kernel(x, W, b, ctx_mod, context):
    xf = jnp.reshape(x, (T, L))
    b2 = jnp.reshape(b, (1, L))
    cm8 = jnp.concatenate(
        [ctx_mod, jnp.broadcast_to(ctx_mod[:1], (8 - ctx_mod.shape[0], L))],
        axis=0)

    x_out, argm = pl.pallas_call(
        _tc_body,
        grid=(GRID,),
        in_specs=[
            pl.BlockSpec((TB, L), lambda i: ((i % N_RECEP) * NBB + i // N_RECEP, 0)),
            pl.BlockSpec((C, L), lambda i: (0, 0)),
            pl.BlockSpec((L, L), lambda i: (0, 0)),
            pl.BlockSpec((1, L), lambda i: (0, 0)),
            pl.BlockSpec((8, L), lambda i: (0, 0)),
        ],
        out_specs=[
            pl.BlockSpec((TB, L), lambda i: (i // N_RECEP, 0)),
            pl.BlockSpec((TB, 1), lambda i: ((i % N_RECEP) * NBB + i // N_RECEP, 0)),
        ],
        out_shape=[
            jax.ShapeDtypeStruct((BATCH, L), jnp.float32),
            jax.ShapeDtypeStruct((T, 1), jnp.int32),
        ],
        scratch_shapes=[
            pltpu.VMEM((C, L), jnp.float32),
            pltpu.VMEM((1, C), jnp.float32),
        ],
        compiler_params=pltpu.CompilerParams(
            dimension_semantics=("arbitrary",)),
    )(xf, context, W, b2, cm8)

    new_context = _sc_update(context, xf, argm[:, 0])
    return (x_out, new_context)


# speedup vs baseline: 1.4049x; 1.1345x over previous
import functools

import jax
import jax.numpy as jnp
from jax import lax
from jax.experimental import pallas as pl
from jax.experimental.pallas import tpu as pltpu
from jax.experimental.pallas import tpu_sc as plsc

N_RECEP = 4
BATCH = 2048
L = 128
C = 4096
T = N_RECEP * BATCH
AVG_N = 50000.0

TB = 512
NBB = BATCH // TB
GRID = NBB * N_RECEP

NC = 2
NS = 16
NW = NC * NS
ROWS_PER_W = C // NW
TOK_PER_W = T // NW
CHUNK = 128
NCHUNK = TOK_PER_W // CHUNK


def _prep_body(ctx_ref, cm_ref, cn_ref, mseg_ref):
    ctx = ctx_ref[...]
    cnorm = jnp.sqrt(jnp.sum(ctx * ctx, axis=1, keepdims=True))
    cn_ref[...] = ctx / (cnorm + 1e-8)
    seg = lax.dot_general(ctx, cm_ref[...], (((1,), (1,)), ((), ())),
                          preferred_element_type=jnp.float32)
    mseg_ref[...] = jnp.max(seg, axis=1, keepdims=True)


def _tc_body(x_ref, cn_ref, w_ref, b_ref, mseg_ref, out_ref, argm_ref):
    i = pl.program_id(0)
    n = i % N_RECEP

    xs = x_ref[...]
    xnorm = jnp.sqrt(jnp.sum(xs * xs, axis=1, keepdims=True))
    xn = xs / (xnorm + 1e-8)
    sim = lax.dot_general(xn, cn_ref[...], (((1,), (1,)), ((), ())),
                          preferred_element_type=jnp.float32)
    m = jnp.max(sim, axis=1, keepdims=True)
    iota1 = lax.broadcasted_iota(jnp.int32, (1, C), 1)
    idxm = jnp.where(sim == m, iota1, jnp.int32(2**30))
    am = jnp.min(idxm, axis=1, keepdims=True)
    row = lax.shift_right_logical(am, 7)
    col = jnp.bitwise_and(am, jnp.int32(L - 1))
    oh_row = (lax.broadcasted_iota(jnp.int32, (1, C // L), 1) == row
              ).astype(jnp.float32)
    t1 = lax.dot_general(oh_row, mseg_ref[...], (((1,), (0,)), ((), ())),
                         preferred_element_type=jnp.float32)
    oh_col = (lax.broadcasted_iota(jnp.int32, (1, L), 1) == col
              ).astype(jnp.float32)
    msel = jnp.sum(t1 * oh_col, axis=1, keepdims=True)
    act = jax.nn.sigmoid(msel)

    rec = jax.nn.gelu(lax.dot_general(xs, w_ref[...], (((1,), (0,)), ((), ())),
                                      preferred_element_type=jnp.float32)
                      + b_ref[...])
    contrib = rec * act * (1.0 / N_RECEP)

    @pl.when(n == 0)
    def _set():
        out_ref[...] = contrib

    @pl.when(n != 0)
    def _acc():
        out_ref[...] += contrib

    argm_ref[...] = am


def _sc_body(ctx_hbm, xs_hbm, idx_hbm, out_hbm, idxv, rows, xsv, sem):
    wid = lax.axis_index("s") * NC + lax.axis_index("c")
    rbase = wid * ROWS_PER_W
    pltpu.sync_copy(ctx_hbm.at[pl.ds(rbase, ROWS_PER_W)], rows)
    pltpu.sync_copy(rows, out_hbm.at[pl.ds(rbase, ROWS_PER_W)])
    plsc.subcore_barrier()
    for j in range(NCHUNK):
        tbase = wid * TOK_PER_W + j * CHUNK
        pltpu.sync_copy(idx_hbm.at[pl.ds(tbase, CHUNK)], idxv)
        pltpu.async_copy(ctx_hbm.at[idxv], rows, sem).wait()
        pltpu.sync_copy(xs_hbm.at[pl.ds(tbase, CHUNK)], xsv)

        def body(r, carry):
            for cc in range(L // 16):
                sl = pl.ds(cc * 16, 16)
                old = rows[r, sl]
                rows[r, sl] = (old * (AVG_N - 1.0) + xsv[r, sl]) / AVG_N
            return carry

        lax.fori_loop(0, CHUNK, body, 0)
        pltpu.async_copy(rows, out_hbm.at[idxv], sem).wait()


def _sc_update(context, xf, idx):
    sc = functools.partial(
        pl.kernel,
        out_type=jax.ShapeDtypeStruct((C, L), jnp.float32),
        mesh=plsc.VectorSubcoreMesh(core_axis_name="c", subcore_axis_name="s",
                                    num_cores=NC, num_subcores=NS),
        scratch_types=[
            pltpu.VMEM((CHUNK,), jnp.int32),
            pltpu.VMEM((CHUNK, L), jnp.float32),
            pltpu.VMEM((CHUNK, L), jnp.float32),
            pltpu.SemaphoreType.DMA,
        ],
    )(_sc_body)
    return sc(context, xf, idx)


def kernel(x, W, b, ctx_mod, context):
    xf = jnp.reshape(x, (T, L))
    b2 = jnp.reshape(b, (1, L))
    cm8 = jnp.concatenate(
        [ctx_mod, jnp.broadcast_to(ctx_mod[:1], (8 - ctx_mod.shape[0], L))],
        axis=0)

    cn, mseg_col = pl.pallas_call(
        _prep_body,
        out_shape=[
            jax.ShapeDtypeStruct((C, L), jnp.float32),
            jax.ShapeDtypeStruct((C, 1), jnp.float32),
        ],
    )(context, cm8)
    mseg2d = jnp.reshape(mseg_col, (C // L, L))

    x_out, argm = pl.pallas_call(
        _tc_body,
        grid=(GRID,),
        in_specs=[
            pl.BlockSpec((TB, L), lambda i: ((i % N_RECEP) * NBB + i // N_RECEP, 0)),
            pl.BlockSpec((C, L), lambda i: (0, 0)),
            pl.BlockSpec((L, L), lambda i: (0, 0)),
            pl.BlockSpec((1, L), lambda i: (0, 0)),
            pl.BlockSpec((C // L, L), lambda i: (0, 0)),
        ],
        out_specs=[
            pl.BlockSpec((TB, L), lambda i: (i // N_RECEP, 0)),
            pl.BlockSpec((TB, 1), lambda i: ((i % N_RECEP) * NBB + i // N_RECEP, 0)),
        ],
        out_shape=[
            jax.ShapeDtypeStruct((BATCH, L), jnp.float32),
            jax.ShapeDtypeStruct((T, 1), jnp.int32),
        ],
        compiler_params=pltpu.CompilerParams(
            dimension_semantics=("arbitrary",)),
    )(xf, cn, W, b2, mseg2d)

    new_context = _sc_update(context, xf, argm[:, 0])
    return (x_out, new_context)


# speedup vs baseline: 1.5803x; 1.1248x over previous
import functools

import jax
import jax.numpy as jnp
from jax import lax
from jax.experimental import pallas as pl
from jax.experimental.pallas import tpu as pltpu
from jax.experimental.pallas import tpu_sc as plsc

N_RECEP = 4
BATCH = 2048
L = 128
C = 4096
T = N_RECEP * BATCH
AVG_N = 50000.0

TB = 512
NBB = BATCH // TB
GRID = NBB * N_RECEP

NC = 2
NS = 16
NW = NC * NS
ROWS_PER_W = C // NW
TOK_PER_W = T // NW
CHUNK = 128
NCHUNK = TOK_PER_W // CHUNK


def _prep_body(ctx_ref, cm_ref, cn_ref, mseg_ref):
    ctx = ctx_ref[...]
    cnorm = jnp.sqrt(jnp.sum(ctx * ctx, axis=1, keepdims=True))
    cn_ref[...] = ctx / (cnorm + 1e-8)
    seg = lax.dot_general(ctx, cm_ref[...], (((1,), (1,)), ((), ())),
                          preferred_element_type=jnp.float32)
    mseg = jnp.max(seg, axis=1, keepdims=True)
    mseg_ref[...] = jnp.reshape(mseg, (C // L, L))


def _tc_body(x_ref, cn_ref, w_ref, b_ref, mseg_ref, out_ref, argm_ref):
    i = pl.program_id(0)
    n = i % N_RECEP

    xs = x_ref[...]
    xnorm = jnp.sqrt(jnp.sum(xs * xs, axis=1, keepdims=True))
    xn = xs / (xnorm + 1e-8)
    sim = lax.dot_general(xn, cn_ref[...], (((1,), (1,)), ((), ())),
                          preferred_element_type=jnp.float32)
    m = jnp.max(sim, axis=1, keepdims=True)
    iota1 = lax.broadcasted_iota(jnp.int32, (1, C), 1)
    idxm = jnp.where(sim == m, iota1, jnp.int32(2**30))
    am = jnp.min(idxm, axis=1, keepdims=True)
    row = lax.shift_right_logical(am, 7)
    col = jnp.bitwise_and(am, jnp.int32(L - 1))
    oh_row = (lax.broadcasted_iota(jnp.int32, (1, C // L), 1) == row
              ).astype(jnp.float32)
    t1 = lax.dot_general(oh_row, mseg_ref[...], (((1,), (0,)), ((), ())),
                         preferred_element_type=jnp.float32)
    oh_col = (lax.broadcasted_iota(jnp.int32, (1, L), 1) == col
              ).astype(jnp.float32)
    msel = jnp.sum(t1 * oh_col, axis=1, keepdims=True)
    act = jax.nn.sigmoid(msel)

    rec = jax.nn.gelu(lax.dot_general(xs, w_ref[...], (((1,), (0,)), ((), ())),
                                      preferred_element_type=jnp.float32)
                      + b_ref[...])
    contrib = rec * act * (1.0 / N_RECEP)

    @pl.when(n == 0)
    def _set():
        out_ref[...] = contrib

    @pl.when(n != 0)
    def _acc():
        out_ref[...] += contrib

    tblk = n * NBB + i // N_RECEP
    argm_ref[pl.ds(tblk * (TB // L), TB // L), :] = jnp.reshape(am, (TB // L, L))


def _sc_body(ctx_hbm, xs_hbm, idx_hbm, out_hbm, idxv, rows, xsv, cpv, sem):
    wid = lax.axis_index("s") * NC + lax.axis_index("c")
    pltpu.sync_copy(idx_hbm.at[wid * NCHUNK], idxv)
    gather = pltpu.async_copy(ctx_hbm.at[idxv], rows, sem)
    rbase = wid * ROWS_PER_W
    pltpu.sync_copy(ctx_hbm.at[pl.ds(rbase, ROWS_PER_W)], cpv)
    pltpu.sync_copy(cpv, out_hbm.at[pl.ds(rbase, ROWS_PER_W)])
    gather.wait()
    plsc.subcore_barrier()
    for j in range(NCHUNK):
        tbase = wid * TOK_PER_W + j * CHUNK
        pltpu.sync_copy(xs_hbm.at[pl.ds(tbase, CHUNK)], xsv)

        def body(r, carry):
            for cc in range(L // 16):
                sl = pl.ds(cc * 16, 16)
                old = rows[r, sl]
                rows[r, sl] = (old * (AVG_N - 1.0) + xsv[r, sl]) * (1.0 / AVG_N)
            return carry

        lax.fori_loop(0, CHUNK, body, 0)
        pltpu.async_copy(rows, out_hbm.at[idxv], sem).wait()
        if j + 1 < NCHUNK:
            pltpu.sync_copy(idx_hbm.at[wid * NCHUNK + j + 1], idxv)
            pltpu.async_copy(ctx_hbm.at[idxv], rows, sem).wait()


def _sc_update(context, xf, idx):
    sc = functools.partial(
        pl.kernel,
        out_type=jax.ShapeDtypeStruct((C, L), jnp.float32),
        mesh=plsc.VectorSubcoreMesh(core_axis_name="c", subcore_axis_name="s",
                                    num_cores=NC, num_subcores=NS),
        scratch_types=[
            pltpu.VMEM((CHUNK,), jnp.int32),
            pltpu.VMEM((CHUNK, L), jnp.float32),
            pltpu.VMEM((CHUNK, L), jnp.float32),
            pltpu.VMEM((ROWS_PER_W, L), jnp.float32),
            pltpu.SemaphoreType.DMA,
        ],
    )(_sc_body)
    return sc(context, xf, idx)


def kernel(x, W, b, ctx_mod, context):
    xf = jnp.reshape(x, (T, L))
    b2 = jnp.reshape(b, (1, L))

    cn, mseg2d = pl.pallas_call(
        _prep_body,
        out_shape=[
            jax.ShapeDtypeStruct((C, L), jnp.float32),
            jax.ShapeDtypeStruct((C // L, L), jnp.float32),
        ],
    )(context, ctx_mod)

    x_out, argm = pl.pallas_call(
        _tc_body,
        grid=(GRID,),
        in_specs=[
            pl.BlockSpec((TB, L), lambda i: ((i % N_RECEP) * NBB + i // N_RECEP, 0)),
            pl.BlockSpec((C, L), lambda i: (0, 0)),
            pl.BlockSpec((L, L), lambda i: (0, 0)),
            pl.BlockSpec((1, L), lambda i: (0, 0)),
            pl.BlockSpec((C // L, L), lambda i: (0, 0)),
        ],
        out_specs=[
            pl.BlockSpec((TB, L), lambda i: (i // N_RECEP, 0)),
            pl.BlockSpec((T // L, L), lambda i: (0, 0)),
        ],
        out_shape=[
            jax.ShapeDtypeStruct((BATCH, L), jnp.float32),
            jax.ShapeDtypeStruct((T // L, L), jnp.int32),
        ],
        compiler_params=pltpu.CompilerParams(
            dimension_semantics=("arbitrary",)),
    )(xf, cn, W, b2, mseg2d)

    new_context = _sc_update(context, xf, argm)
    return (x_out, new_context)


# speedup vs baseline: 1.6896x; 1.0692x over previous
import functools

import jax
import jax.numpy as jnp
from jax import lax
from jax.experimental import pallas as pl
from jax.experimental.pallas import tpu as pltpu
from jax.experimental.pallas import tpu_sc as plsc

N_RECEP = 4
BATCH = 2048
L = 128
C = 4096
T = N_RECEP * BATCH
AVG_N = 50000.0

TB = 512
NBB = BATCH // TB
GRID = NBB * N_RECEP

NC = 2
NS = 16
NW = NC * NS
ROWS_PER_W = C // NW
TOK_PER_W = T // NW
CHUNK = 128
NCHUNK = TOK_PER_W // CHUNK


def _prep_body(ctx_ref, cm_ref, cn_ref, mseg_ref):
    ctx = ctx_ref[...]
    cnorm = jnp.sqrt(jnp.sum(ctx * ctx, axis=1, keepdims=True))
    cn_ref[...] = ctx / (cnorm + 1e-8)
    seg = lax.dot_general(ctx, cm_ref[...], (((1,), (1,)), ((), ())),
                          preferred_element_type=jnp.float32)
    mseg = jnp.max(seg, axis=1, keepdims=True)
    mseg_ref[...] = jnp.reshape(mseg, (C // L, L))


def _tc_a_body(x_ref, cn_ref, mseg_ref, act_ref, argm_ref):
    i = pl.program_id(0)

    xs = x_ref[...]
    xnorm = jnp.sqrt(jnp.sum(xs * xs, axis=1, keepdims=True))
    xn = xs / (xnorm + 1e-8)
    sim = lax.dot_general(xn, cn_ref[...], (((1,), (1,)), ((), ())),
                          preferred_element_type=jnp.float32)
    m = jnp.max(sim, axis=1, keepdims=True)
    iota1 = lax.broadcasted_iota(jnp.int32, (1, C), 1).astype(jnp.float32)
    idxm = jnp.where(sim == m, iota1, jnp.float32(2.0**30))
    amf = jnp.min(idxm, axis=1, keepdims=True)
    am = amf.astype(jnp.int32)
    row = lax.shift_right_logical(am, 7)
    col = jnp.bitwise_and(am, jnp.int32(L - 1))
    oh_row = (lax.broadcasted_iota(jnp.int32, (1, C // L), 1) == row
              ).astype(jnp.float32)
    t1 = lax.dot_general(oh_row, mseg_ref[...], (((1,), (0,)), ((), ())),
                         preferred_element_type=jnp.float32)
    oh_col = (lax.broadcasted_iota(jnp.int32, (1, L), 1) == col
              ).astype(jnp.float32)
    msel = jnp.sum(t1 * oh_col, axis=1, keepdims=True)
    act_ref[...] = jax.nn.sigmoid(msel)

    argm_ref[pl.ds(i * (TB // L), TB // L), :] = jnp.reshape(am, (TB // L, L))


def _tc_b_body(x_ref, w_ref, b_ref, act_ref, out_ref):
    i = pl.program_id(0)
    n = i % N_RECEP

    xs = x_ref[...]
    rec = jax.nn.gelu(lax.dot_general(xs, w_ref[...], (((1,), (0,)), ((), ())),
                                      preferred_element_type=jnp.float32)
                      + b_ref[...])
    contrib = rec * act_ref[...] * (1.0 / N_RECEP)

    @pl.when(n == 0)
    def _set():
        out_ref[...] = contrib

    @pl.when(n != 0)
    def _acc():
        out_ref[...] += contrib


def _sc_body(ctx_hbm, xs_hbm, idx_hbm, out_hbm, idxv, rows, xsv, cpv, sem):
    wid = lax.axis_index("s") * NC + lax.axis_index("c")
    pltpu.sync_copy(idx_hbm.at[wid * NCHUNK], idxv)
    gather = pltpu.async_copy(ctx_hbm.at[idxv], rows, sem)
    rbase = wid * ROWS_PER_W
    pltpu.sync_copy(ctx_hbm.at[pl.ds(rbase, ROWS_PER_W)], cpv)
    pltpu.sync_copy(cpv, out_hbm.at[pl.ds(rbase, ROWS_PER_W)])
    gather.wait()
    plsc.subcore_barrier()
    for j in range(NCHUNK):
        tbase = wid * TOK_PER_W + j * CHUNK
        pltpu.sync_copy(xs_hbm.at[pl.ds(tbase, CHUNK)], xsv)

        def body(r, carry):
            for cc in range(L // 16):
                sl = pl.ds(cc * 16, 16)
                old = rows[r, sl]
                rows[r, sl] = (old * (AVG_N - 1.0) + xsv[r, sl]) * (1.0 / AVG_N)
            return carry

        lax.fori_loop(0, CHUNK, body, 0)
        pltpu.async_copy(rows, out_hbm.at[idxv], sem).wait()
        if j + 1 < NCHUNK:
            pltpu.sync_copy(idx_hbm.at[wid * NCHUNK + j + 1], idxv)
            pltpu.async_copy(ctx_hbm.at[idxv], rows, sem).wait()


def _sc_update(context, xf, idx):
    sc = functools.partial(
        pl.kernel,
        out_type=jax.ShapeDtypeStruct((C, L), jnp.float32),
        mesh=plsc.VectorSubcoreMesh(core_axis_name="c", subcore_axis_name="s",
                                    num_cores=NC, num_subcores=NS),
        scratch_types=[
            pltpu.VMEM((CHUNK,), jnp.int32),
            pltpu.VMEM((CHUNK, L), jnp.float32),
            pltpu.VMEM((CHUNK, L), jnp.float32),
            pltpu.VMEM((ROWS_PER_W, L), jnp.float32),
            pltpu.SemaphoreType.DMA,
        ],
    )(_sc_body)
    return sc(context, xf, idx)


def kernel(x, W, b, ctx_mod, context):
    xf = jnp.reshape(x, (T, L))
    b2 = jnp.reshape(b, (1, L))

    cn, mseg2d = pl.pallas_call(
        _prep_body,
        out_shape=[
            jax.ShapeDtypeStruct((C, L), jnp.float32),
            jax.ShapeDtypeStruct((C // L, L), jnp.float32),
        ],
    )(context, ctx_mod)

    act, argm = pl.pallas_call(
        _tc_a_body,
        grid=(GRID,),
        in_specs=[
            pl.BlockSpec((TB, L), lambda i: (i, 0)),
            pl.BlockSpec((C, L), lambda i: (0, 0)),
            pl.BlockSpec((C // L, L), lambda i: (0, 0)),
        ],
        out_specs=[
            pl.BlockSpec((TB, 1), lambda i: (i, 0)),
            pl.BlockSpec((T // L, L), lambda i: (0, 0)),
        ],
        out_shape=[
            jax.ShapeDtypeStruct((T, 1), jnp.float32),
            jax.ShapeDtypeStruct((T // L, L), jnp.int32),
        ],
        compiler_params=pltpu.CompilerParams(
            dimension_semantics=("arbitrary",)),
    )(xf, cn, mseg2d)

    new_context = _sc_update(context, xf, argm)

    x_out = pl.pallas_call(
        _tc_b_body,
        grid=(GRID,),
        in_specs=[
            pl.BlockSpec((TB, L), lambda i: ((i % N_RECEP) * NBB + i // N_RECEP, 0)),
            pl.BlockSpec((L, L), lambda i: (0, 0)),
            pl.BlockSpec((1, L), lambda i: (0, 0)),
            pl.BlockSpec((TB, 1), lambda i: ((i % N_RECEP) * NBB + i // N_RECEP, 0)),
        ],
        out_specs=pl.BlockSpec((TB, L), lambda i: (i // N_RECEP, 0)),
        out_shape=jax.ShapeDtypeStruct((BATCH, L), jnp.float32),
        compiler_params=pltpu.CompilerParams(
            dimension_semantics=("arbitrary",)),
    )(xf, W, b2, act)

    return (x_out, new_context)


# speedup vs baseline: 1.7148x; 1.0149x over previous
import functools

import jax
import jax.numpy as jnp
from jax import lax
from jax.experimental import pallas as pl
from jax.experimental.pallas import tpu as pltpu
from jax.experimental.pallas import tpu_sc as plsc

N_RECEP = 4
BATCH = 2048
L = 128
C = 4096
T = N_RECEP * BATCH
AVG_N = 50000.0

TB = 512
NBB = BATCH // TB
GRID = NBB * N_RECEP

NC = 2
NS = 16
NW = NC * NS
ROWS_PER_W = C // NW
TOK_PER_W = T // NW
CHUNK = 128
NCHUNK = TOK_PER_W // CHUNK


def _prep_body(ctx_ref, cm_ref, cn_ref, mseg_ref):
    ctx = ctx_ref[...]
    cnorm = jnp.sqrt(jnp.sum(ctx * ctx, axis=1, keepdims=True))
    cn_ref[...] = ctx / (cnorm + 1e-8)
    seg = lax.dot_general(ctx, cm_ref[...], (((1,), (1,)), ((), ())),
                          preferred_element_type=jnp.float32)
    mseg = jnp.max(seg, axis=1, keepdims=True)
    mseg_ref[...] = jnp.reshape(mseg, (C // L, L))


def _tc_a_body(x_ref, cn_ref, mseg_ref, act_ref, argm_ref):
    i = pl.program_id(0)

    xs = x_ref[...]
    xnorm = jnp.sqrt(jnp.sum(xs * xs, axis=1, keepdims=True))
    xn = xs / (xnorm + 1e-8)
    sim = lax.dot_general(xn, cn_ref[...], (((1,), (1,)), ((), ())),
                          preferred_element_type=jnp.float32)
    m = jnp.max(sim, axis=1, keepdims=True)
    iota1 = lax.broadcasted_iota(jnp.int32, (1, C), 1).astype(jnp.float32)
    idxm = jnp.where(sim == m, iota1, jnp.float32(2.0**30))
    amf = jnp.min(idxm, axis=1, keepdims=True)
    am = amf.astype(jnp.int32)
    row = lax.shift_right_logical(am, 7)
    col = jnp.bitwise_and(am, jnp.int32(L - 1))
    oh_row = (lax.broadcasted_iota(jnp.int32, (1, C // L), 1) == row
              ).astype(jnp.float32)
    t1 = lax.dot_general(oh_row, mseg_ref[...], (((1,), (0,)), ((), ())),
                         preferred_element_type=jnp.float32)
    oh_col = (lax.broadcasted_iota(jnp.int32, (1, L), 1) == col
              ).astype(jnp.float32)
    msel = jnp.sum(t1 * oh_col, axis=1, keepdims=True)
    act = jax.nn.sigmoid(msel)

    sl = pl.ds(i * (TB // L), TB // L)
    act_ref[sl, :] = jnp.reshape(act, (TB // L, L))
    argm_ref[sl, :] = jnp.reshape(am, (TB // L, L))


def _tc_b_body(x_ref, w_ref, b_ref, act_ref, out_ref):
    i = pl.program_id(0)
    n = i % N_RECEP

    xs = x_ref[...]
    rec = jax.nn.gelu(lax.dot_general(xs, w_ref[...], (((1,), (0,)), ((), ())),
                                      preferred_element_type=jnp.float32)
                      + b_ref[...])
    tblk = n * NBB + i // N_RECEP
    actblk = act_ref[pl.ds(tblk * (TB // L), TB // L), :]
    ids = lax.broadcasted_iota(jnp.int32, (TB, 1), 0)
    oh_r = (lax.broadcasted_iota(jnp.int32, (1, TB // L), 1)
            == lax.shift_right_logical(ids, 7)).astype(jnp.float32)
    t1 = lax.dot_general(oh_r, actblk, (((1,), (0,)), ((), ())),
                         preferred_element_type=jnp.float32)
    oh_c = (lax.broadcasted_iota(jnp.int32, (1, L), 1)
            == jnp.bitwise_and(ids, jnp.int32(L - 1))).astype(jnp.float32)
    act = jnp.sum(t1 * oh_c, axis=1, keepdims=True)
    contrib = rec * act * (1.0 / N_RECEP)

    @pl.when(n == 0)
    def _set():
        out_ref[...] = contrib

    @pl.when(n != 0)
    def _acc():
        out_ref[...] += contrib


def _sc_body(ctx_hbm, xs_hbm, idx_hbm, out_hbm,
             idx0, idx1, rows0, rows1, xsv0, xsv1, cpv,
             sem0, sem1, semx0, semx1, sems):
    wid = lax.axis_index("s") * NC + lax.axis_index("c")
    tbase = wid * TOK_PER_W
    pltpu.sync_copy(idx_hbm.at[wid * NCHUNK], idx0)
    pltpu.sync_copy(idx_hbm.at[wid * NCHUNK + 1], idx1)
    g0 = pltpu.async_copy(ctx_hbm.at[idx0], rows0, sem0)
    g1 = pltpu.async_copy(ctx_hbm.at[idx1], rows1, sem1)
    x0 = pltpu.async_copy(xs_hbm.at[pl.ds(tbase, CHUNK)], xsv0, semx0)
    x1 = pltpu.async_copy(xs_hbm.at[pl.ds(tbase + CHUNK, CHUNK)], xsv1, semx1)
    rbase = wid * ROWS_PER_W
    pltpu.sync_copy(ctx_hbm.at[pl.ds(rbase, ROWS_PER_W)], cpv)
    pltpu.sync_copy(cpv, out_hbm.at[pl.ds(rbase, ROWS_PER_W)])
    plsc.subcore_barrier()

    def _avg(rows, xsv):
        def body(r, carry):
            for cc in range(L // 16):
                sl = pl.ds(cc * 16, 16)
                old = rows[r, sl]
                rows[r, sl] = (old * (AVG_N - 1.0) + xsv[r, sl]) * (1.0 / AVG_N)
            return carry
        lax.fori_loop(0, CHUNK, body, 0)

    g0.wait()
    x0.wait()
    _avg(rows0, xsv0)
    s0 = pltpu.async_copy(rows0, out_hbm.at[idx0], sems)
    g1.wait()
    x1.wait()
    _avg(rows1, xsv1)
    s0.wait()
    pltpu.async_copy(rows1, out_hbm.at[idx1], sems).wait()


def _sc_update(context, xf, idx):
    sc = functools.partial(
        pl.kernel,
        out_type=jax.ShapeDtypeStruct((C, L), jnp.float32),
        mesh=plsc.VectorSubcoreMesh(core_axis_name="c", subcore_axis_name="s",
                                    num_cores=NC, num_subcores=NS),
        scratch_types=[
            pltpu.VMEM((CHUNK,), jnp.int32),
            pltpu.VMEM((CHUNK,), jnp.int32),
            pltpu.VMEM((CHUNK, L), jnp.float32),
            pltpu.VMEM((CHUNK, L), jnp.float32),
            pltpu.VMEM((CHUNK, L), jnp.float32),
            pltpu.VMEM((CHUNK, L), jnp.float32),
            pltpu.VMEM((ROWS_PER_W, L), jnp.float32),
            pltpu.SemaphoreType.DMA,
            pltpu.SemaphoreType.DMA,
            pltpu.SemaphoreType.DMA,
            pltpu.SemaphoreType.DMA,
            pltpu.SemaphoreType.DMA,
        ],
    )(_sc_body)
    return sc(context, xf, idx)


def kernel(x, W, b, ctx_mod, context):
    xf = jnp.reshape(x, (T, L))
    b2 = jnp.reshape(b, (1, L))

    cn, mseg2d = pl.pallas_call(
        _prep_body,
        out_shape=[
            jax.ShapeDtypeStruct((C, L), jnp.float32),
            jax.ShapeDtypeStruct((C // L, L), jnp.float32),
        ],
    )(context, ctx_mod)

    act, argm = pl.pallas_call(
        _tc_a_body,
        grid=(GRID,),
        in_specs=[
            pl.BlockSpec((TB, L), lambda i: (i, 0)),
            pl.BlockSpec((C, L), lambda i: (0, 0)),
            pl.BlockSpec((C // L, L), lambda i: (0, 0)),
        ],
        out_specs=[
            pl.BlockSpec((T // L, L), lambda i: (0, 0)),
            pl.BlockSpec((T // L, L), lambda i: (0, 0)),
        ],
        out_shape=[
            jax.ShapeDtypeStruct((T // L, L), jnp.float32),
            jax.ShapeDtypeStruct((T // L, L), jnp.int32),
        ],
        compiler_params=pltpu.CompilerParams(
            dimension_semantics=("arbitrary",)),
    )(xf, cn, mseg2d)

    new_context = _sc_update(context, xf, argm)

    x_out = pl.pallas_call(
        _tc_b_body,
        grid=(GRID,),
        in_specs=[
            pl.BlockSpec((TB, L), lambda i: ((i % N_RECEP) * NBB + i // N_RECEP, 0)),
            pl.BlockSpec((L, L), lambda i: (0, 0)),
            pl.BlockSpec((1, L), lambda i: (0, 0)),
            pl.BlockSpec((T // L, L), lambda i: (0, 0)),
        ],
        out_specs=pl.BlockSpec((TB, L), lambda i: (i // N_RECEP, 0)),
        out_shape=jax.ShapeDtypeStruct((BATCH, L), jnp.float32),
        compiler_params=pltpu.CompilerParams(
            dimension_semantics=("arbitrary",)),
    )(xf, W, b2, act)

    return (x_out, new_context)
